# baseline (speedup 1.0000x reference)
import jax
import jax.numpy as jnp
from jax.experimental import pallas as pl

PRE_NMS_THRESH = 0.05
PRE_NMS_TOP_N = 1000
NMS_THRESH = 0.6
FPN_POST_NMS_TOP_N = 100


def _permute_and_flatten(t, N, A, C, H, W):
    t = t.reshape(N, A, C, H, W)
    t = jnp.transpose(t, (0, 3, 4, 1, 2))
    return t.reshape(N, H * W * A, C)


def _distance2bbox(points, distance):
    x1 = points[..., 0] - distance[..., 0]
    y1 = points[..., 1] - distance[..., 1]
    x2 = points[..., 0] + distance[..., 2]
    y2 = points[..., 1] + distance[..., 3]
    return jnp.stack([x1, y1, x2, y2], axis=-1)


def _nms_single(boxes, scores, labels, valid):
    off = labels.astype(jnp.float32) * 100000.0
    b = boxes + off[:, None]
    areas = (b[:, 2] - b[:, 0]) * (b[:, 3] - b[:, 1])
    lt = jnp.maximum(b[:, None, :2], b[None, :, :2])
    rb = jnp.minimum(b[:, None, 2:], b[None, :, 2:])
    wh = jnp.clip(rb - lt, 0.0, None)
    inter = wh[..., 0] * wh[..., 1]
    iou = inter / (areas[:, None] + areas[None, :] - inter + 1e-9)
    s0 = jnp.where(valid, scores, -1.0)

    def step(s_cur, _):
        i = jnp.argmax(s_cur)
        kept = s_cur[i] > 0.0
        sup = iou[i] > NMS_THRESH
        s_new = jnp.where(sup, -1.0, s_cur).at[i].set(-1.0)
        return s_new, (i, kept)

    _, (kept_idx, kept_mask) = jax.lax.scan(step, s0, None, length=FPN_POST_NMS_TOP_N)
    return kept_idx, kept_mask


def _mask_kernel(x_ref, m_ref, o_ref):
    o_ref[...] = x_ref[...] * m_ref[...]


def kernel(box_cls, box_regression, centerness, level_points, img_sizes):
    N, AC, H, W = box_cls.shape
    A = box_regression.shape[1] // 4
    C = AC // A
    cls = jax.nn.sigmoid(_permute_and_flatten(box_cls, N, A, C, H, W))
    reg = _permute_and_flatten(box_regression, N, A, 4, H, W).reshape(N, -1, 4)
    ctr = jax.nn.sigmoid(_permute_and_flatten(centerness, N, A, 1, H, W).reshape(N, -1))
    cand = cls > PRE_NMS_THRESH
    combined = cls * ctr[:, :, None]
    flat = (combined * cand).reshape(N, -1)
    vals, idx = jax.lax.top_k(flat, PRE_NMS_TOP_N)
    loc = idx // C
    labels = idx % C + 1
    valid = jnp.take_along_axis(cand.reshape(N, -1), idx, axis=1)
    pts = level_points[loc]
    dist = jnp.take_along_axis(reg, loc[:, :, None], axis=1)
    boxes = _distance2bbox(pts, dist)
    wmax = (img_sizes[:, 0].astype(jnp.float32) - 1.0)[:, None]
    hmax = (img_sizes[:, 1].astype(jnp.float32) - 1.0)[:, None]
    x1 = jnp.clip(boxes[..., 0], 0.0, wmax)
    y1 = jnp.clip(boxes[..., 1], 0.0, hmax)
    x2 = jnp.clip(boxes[..., 2], 0.0, wmax)
    y2 = jnp.clip(boxes[..., 3], 0.0, hmax)
    boxes = jnp.stack([x1, y1, x2, y2], axis=-1)
    valid = valid & (x2 - x1 >= 0) & (y2 - y1 >= 0)
    scores = jnp.sqrt(jnp.clip(vals, 1e-12, None))
    kept_idx, kept_mask = jax.vmap(_nms_single)(boxes, scores, labels, valid)
    boxes_k = jnp.take_along_axis(boxes, kept_idx[:, :, None], axis=1)
    scores_k = jnp.take_along_axis(scores, kept_idx, axis=1)
    labels_k = jnp.take_along_axis(labels, kept_idx, axis=1)
    out = jnp.concatenate([boxes_k, scores_k[:, :, None], labels_k[:, :, None].astype(jnp.float32)], axis=-1)
    mask = jnp.broadcast_to(kept_mask[:, :, None].astype(jnp.float32), out.shape)
    out = pl.pallas_call(
        _mask_kernel,
        out_shape=jax.ShapeDtypeStruct(out.shape, out.dtype),
    )(out, mask)
    return out


# speedup vs baseline: 1.0375x; 1.0375x over previous
import functools

import jax
import jax.numpy as jnp
from jax.experimental import pallas as pl

PRE_NMS_THRESH = 0.05
PRE_NMS_TOP_N = 1000
NMS_THRESH = 0.6
FPN_POST_NMS_TOP_N = 100

_PAD = 1024


def _permute_and_flatten(t, N, A, C, H, W):
    t = t.reshape(N, A, C, H, W)
    t = jnp.transpose(t, (0, 3, 4, 1, 2))
    return t.reshape(N, H * W * A, C)


def _nms_kernel(feats_ref, out_ref):
    x1 = feats_ref[0, 0]
    y1 = feats_ref[0, 1]
    x2 = feats_ref[0, 2]
    y2 = feats_ref[0, 3]
    score = feats_ref[0, 4]
    lab = feats_ref[0, 5]
    valid = feats_ref[0, 6]

    off = lab * 100000.0
    xo1 = x1 + off
    yo1 = y1 + off
    xo2 = x2 + off
    yo2 = y2 + off
    area = (xo2 - xo1) * (yo2 - yo1)
    s0 = jnp.where(valid > 0.0, score, -1.0)

    rowi = jax.lax.broadcasted_iota(jnp.int32, (8, 128), 0)
    lanei = jax.lax.broadcasted_iota(jnp.int32, (8, 128), 1)
    flat = rowi * 128 + lanei

    fiota = jax.lax.broadcasted_iota(jnp.int32, (1, 8), 1)

    def body(t, s):
        m = jnp.max(s)
        eq = s == m
        pos = jnp.min(jnp.where(eq, flat, 1 << 30))
        sel = flat == pos
        kf = jnp.where(m > 0.0, 1.0, 0.0)

        def ext(a):
            return jnp.sum(jnp.where(sel, a, 0.0))

        exo1 = ext(xo1)
        eyo1 = ext(yo1)
        exo2 = ext(xo2)
        eyo2 = ext(yo2)
        ear = ext(area)
        elab = ext(lab)
        esc = ext(score)
        ex1 = ext(x1)
        ey1 = ext(y1)
        ex2 = ext(x2)
        ey2 = ext(y2)

        ltx = jnp.maximum(xo1, exo1)
        lty = jnp.maximum(yo1, eyo1)
        rbx = jnp.minimum(xo2, exo2)
        rby = jnp.minimum(yo2, eyo2)
        w = jnp.clip(rbx - ltx, 0.0, None)
        h = jnp.clip(rby - lty, 0.0, None)
        inter = w * h
        iou = inter / (area + ear - inter + 1e-9)
        s_new = jnp.where((iou > NMS_THRESH) | sel, -1.0, s)

        row = (
            jnp.where(fiota == 0, ex1, 0.0)
            + jnp.where(fiota == 1, ey1, 0.0)
            + jnp.where(fiota == 2, ex2, 0.0)
            + jnp.where(fiota == 3, ey2, 0.0)
            + jnp.where(fiota == 4, esc, 0.0)
            + jnp.where(fiota == 5, elab, 0.0)
        )
        out_ref[0, pl.ds(t, 1), :] = row * kf
        return s_new

    jax.lax.fori_loop(0, FPN_POST_NMS_TOP_N, body, s0)


@functools.partial(jax.jit, static_argnames=())
def kernel(box_cls, box_regression, centerness, level_points, img_sizes):
    N, AC, H, W = box_cls.shape
    A = box_regression.shape[1] // 4
    C = AC // A
    cls = jax.nn.sigmoid(_permute_and_flatten(box_cls, N, A, C, H, W))
    reg = _permute_and_flatten(box_regression, N, A, 4, H, W).reshape(N, -1, 4)
    ctr = jax.nn.sigmoid(
        _permute_and_flatten(centerness, N, A, 1, H, W).reshape(N, -1))
    cand = cls > PRE_NMS_THRESH
    combined = cls * ctr[:, :, None]
    flat = (combined * cand).reshape(N, -1)
    vals, idx = jax.lax.top_k(flat, PRE_NMS_TOP_N)
    loc = idx // C
    labels = idx % C + 1
    valid = jnp.take_along_axis(cand.reshape(N, -1), idx, axis=1)
    pts = level_points[loc]
    dist = jnp.take_along_axis(reg, loc[:, :, None], axis=1)
    x1 = pts[..., 0] - dist[..., 0]
    y1 = pts[..., 1] - dist[..., 1]
    x2 = pts[..., 0] + dist[..., 2]
    y2 = pts[..., 1] + dist[..., 3]
    wmax = (img_sizes[:, 0].astype(jnp.float32) - 1.0)[:, None]
    hmax = (img_sizes[:, 1].astype(jnp.float32) - 1.0)[:, None]
    x1 = jnp.clip(x1, 0.0, wmax)
    y1 = jnp.clip(y1, 0.0, hmax)
    x2 = jnp.clip(x2, 0.0, wmax)
    y2 = jnp.clip(y2, 0.0, hmax)
    valid = valid & (x2 - x1 >= 0) & (y2 - y1 >= 0)
    scores = jnp.sqrt(jnp.clip(vals, 1e-12, None))

    feats = jnp.stack(
        [x1, y1, x2, y2, scores, labels.astype(jnp.float32),
         valid.astype(jnp.float32)], axis=1)
    feats = jnp.pad(feats, ((0, 0), (0, 0), (0, _PAD - PRE_NMS_TOP_N)))
    feats = feats.reshape(N, 7, 8, 128)

    out = pl.pallas_call(
        _nms_kernel,
        grid=(N,),
        in_specs=[pl.BlockSpec((1, 7, 8, 128), lambda n: (n, 0, 0, 0))],
        out_specs=pl.BlockSpec((1, 104, 8), lambda n: (n, 0, 0)),
        out_shape=jax.ShapeDtypeStruct((N, 104, 8), jnp.float32),
    )(feats)
    return out[:, :FPN_POST_NMS_TOP_N, :6]
